# baseline (speedup 1.0000x reference)
import functools

import jax
import jax.numpy as jnp
from jax import lax
from jax.experimental import pallas as pl
from jax.experimental.pallas import tpu as pltpu
from jax.experimental.pallas import tpu_sc as plsc

_D = 64
_NC, _NS = 2, 16
_NW = _NC * _NS
_CH = 256
_VL = 16
_GRP = 128


@functools.lru_cache(maxsize=None)
def _build(batch, hist):
    n_tokens = batch * hist
    assert n_tokens % (_NW * _CH * 2) == 0
    rows_per_w = n_tokens // _NW
    n_chunks = rows_per_w // _CH

    mesh = plsc.VectorSubcoreMesh(
        core_axis_name="c", subcore_axis_name="s",
        num_cores=_NC, num_subcores=_NS)

    @functools.partial(
        pl.kernel, mesh=mesh,
        compiler_params=pltpu.CompilerParams(
            use_tc_tiling_on_sc=True, disable_bounds_checks=True),
        out_type=jax.ShapeDtypeStruct((n_tokens, _D), jnp.float32),
        scratch_types=[
            pltpu.VMEM((rows_per_w,), jnp.int32),
            pltpu.VMEM((2, _CH, _D), jnp.float32),
            pltpu.SemaphoreType.DMA,
            pltpu.SemaphoreType.DMA,
            pltpu.SemaphoreType.DMA,
            pltpu.SemaphoreType.DMA,
        ],
    )
    def emb(table_hbm, idx_hbm, out_hbm, idx_v, rows_v, sg0, sg1, ss0, ss1):
        wid = lax.axis_index("s") * _NC + lax.axis_index("c")
        row0 = wid * rows_per_w
        sg = (sg0, sg1)
        ss = (ss0, ss1)

        pltpu.sync_copy(idx_hbm.at[pl.ds(row0, rows_per_w)], idx_v)

        def issue_gathers(g, b):
            def sub(s, carry):
                base = g * _CH + s * _GRP
                vs = [idx_v[pl.ds(base + i * _VL, _VL)]
                      for i in range(_GRP // _VL)]
                for i, v in enumerate(vs):
                    for k in range(_VL):
                        pltpu.async_copy(
                            table_hbm.at[v[k]],
                            rows_v.at[b, s * _GRP + i * _VL + k], sg[b])
                return carry
            lax.fori_loop(0, _CH // _GRP, sub, 0)

        def wait_gathers(b):
            pltpu.make_async_copy(
                rows_v.at[b], out_hbm.at[pl.ds(0, _CH)], sg[b]).wait()

        def issue_store(g, b):
            pltpu.async_copy(
                rows_v.at[b], out_hbm.at[pl.ds(row0 + g * _CH, _CH)], ss[b])

        def wait_store(b):
            pltpu.make_async_copy(
                rows_v.at[b], out_hbm.at[pl.ds(0, _CH)], ss[b]).wait()

        issue_gathers(0, 0)

        def body(g2, carry):
            for b in (0, 1):
                g = 2 * g2 + b
                nb = 1 - b
                wait_gathers(b)
                issue_store(g, b)

                @pl.when(g >= 1)
                def _():
                    wait_store(nb)

                @pl.when(g + 1 < n_chunks)
                def _():
                    issue_gathers(g + 1, nb)
            return carry

        lax.fori_loop(0, n_chunks // 2, body, 0)
        wait_store(1)

    return emb


def kernel(word_vectors, token_ids):
    b, h = token_ids.shape
    idx_flat = token_ids.reshape(-1).astype(jnp.int32)
    out = _build(b, h)(word_vectors, idx_flat)
    return out.reshape(b, h, _D)


# speedup vs baseline: 1.0027x; 1.0027x over previous
import functools

import jax
import jax.numpy as jnp
from jax import lax
from jax.experimental import pallas as pl
from jax.experimental.pallas import tpu as pltpu
from jax.experimental.pallas import tpu_sc as plsc

_D = 64
_NC, _NS = 2, 16
_NW = _NC * _NS
_CH = 256
_VL = 16
_GRP = 64


@functools.lru_cache(maxsize=None)
def _build(batch, hist):
    n_tokens = batch * hist
    assert n_tokens % (_NW * _CH * 2) == 0
    rows_per_w = n_tokens // _NW
    n_chunks = rows_per_w // _CH

    mesh = plsc.VectorSubcoreMesh(
        core_axis_name="c", subcore_axis_name="s",
        num_cores=_NC, num_subcores=_NS)

    @functools.partial(
        pl.kernel, mesh=mesh,
        compiler_params=pltpu.CompilerParams(
            use_tc_tiling_on_sc=True, disable_bounds_checks=True),
        out_type=jax.ShapeDtypeStruct((n_tokens, _D), jnp.float32),
        scratch_types=[
            pltpu.VMEM((rows_per_w,), jnp.int32),
            pltpu.VMEM((2, _CH, _D), jnp.float32),
            pltpu.SemaphoreType.DMA,
            pltpu.SemaphoreType.DMA,
            pltpu.SemaphoreType.DMA,
            pltpu.SemaphoreType.DMA,
        ],
    )
    def emb(table_hbm, idx_hbm, out_hbm, idx_v, rows_v, sg0, sg1, ss0, ss1):
        wid = lax.axis_index("s") * _NC + lax.axis_index("c")
        row0 = wid * rows_per_w
        sg = (sg0, sg1)
        ss = (ss0, ss1)

        pltpu.sync_copy(idx_hbm.at[pl.ds(row0, rows_per_w)], idx_v)

        def issue_gathers(g, b):
            def sub(s, carry):
                base = g * _CH + s * _GRP
                vs = [idx_v[pl.ds(base + i * _VL, _VL)]
                      for i in range(_GRP // _VL)]
                for i, v in enumerate(vs):
                    for k in range(_VL):
                        pltpu.async_copy(
                            table_hbm.at[v[k]],
                            rows_v.at[b, s * _GRP + i * _VL + k], sg[b])
                return carry
            lax.fori_loop(0, _CH // _GRP, sub, 0)

        def wait_gathers(b):
            pltpu.make_async_copy(
                rows_v.at[b], out_hbm.at[pl.ds(0, _CH)], sg[b]).wait()

        def issue_store(g, b):
            pltpu.async_copy(
                rows_v.at[b], out_hbm.at[pl.ds(row0 + g * _CH, _CH)], ss[b])

        def wait_store(b):
            pltpu.make_async_copy(
                rows_v.at[b], out_hbm.at[pl.ds(0, _CH)], ss[b]).wait()

        issue_gathers(0, 0)

        def body(g2, carry):
            for b in (0, 1):
                g = 2 * g2 + b
                nb = 1 - b
                wait_gathers(b)
                issue_store(g, b)

                @pl.when(g >= 1)
                def _():
                    wait_store(nb)

                @pl.when(g + 1 < n_chunks)
                def _():
                    issue_gathers(g + 1, nb)
            return carry

        lax.fori_loop(0, n_chunks // 2, body, 0)
        wait_store(1)

    return emb


def kernel(word_vectors, token_ids):
    b, h = token_ids.shape
    idx_flat = token_ids.reshape(-1).astype(jnp.int32)
    out = _build(b, h)(word_vectors, idx_flat)
    return out.reshape(b, h, _D)
